# baseline (speedup 1.0000x reference)
import functools

import jax
import jax.numpy as jnp
from jax import lax
from jax.experimental import pallas as pl
from jax.experimental.pallas import tpu as pltpu
from jax.experimental.pallas import tpu_sc as plsc

HIDDEN = 64
PADDED = 128
NC = 2
NS = 16
NW = NC * NS
BBLK = 128
L = 16


def kernel(input_ids, token_embedding_weight):
    B, S = input_ids.shape
    table128 = jnp.pad(token_embedding_weight, ((0, 0), (0, PADDED - HIDDEN)))
    idx_t_host = jnp.transpose(input_ids.astype(jnp.int32))

    mesh = plsc.VectorSubcoreMesh(core_axis_name="c", subcore_axis_name="s")

    @functools.partial(
        pl.kernel,
        mesh=mesh,
        out_type=jax.ShapeDtypeStruct((S, HIDDEN, B), jnp.float32),
        compiler_params=pltpu.CompilerParams(needs_layout_passes=False),
        scratch_types=[
            pltpu.VMEM((S, BBLK), jnp.int32),
            pltpu.VMEM((2, BBLK, PADDED), jnp.float32),
            pltpu.VMEM((2, HIDDEN, BBLK), jnp.float32),
            pltpu.SemaphoreType.DMA,
            pltpu.SemaphoreType.DMA,
        ],
    )
    def emb(idx_hbm, table_hbm, out_hbm, idx_t, rows_v, ot_v, gsem, osem):
        wid = lax.axis_index("s") * NC + lax.axis_index("c")
        b0 = wid * BBLK
        pltpu.sync_copy(idx_hbm.at[pl.ds(0, S), pl.ds(b0, BBLK)], idx_t)

        iota = lax.iota(jnp.int32, L)

        def fire(s, p):
            pltpu.async_copy(table_hbm.at[idx_t.at[s]], rows_v.at[p], gsem)

        fire(0, 0)

        def body(s, _):
            p = lax.rem(s, 2)
            pltpu.make_async_copy(
                table_hbm.at[idx_t.at[s]], rows_v.at[p], gsem
            ).wait()

            @pl.when(s + 1 < S)
            def _():
                fire(s + 1, 1 - p)

            @pl.when(s >= 2)
            def _():
                pltpu.make_async_copy(
                    ot_v.at[p],
                    out_hbm.at[s, pl.ds(0, HIDDEN), pl.ds(b0, BBLK)],
                    osem,
                ).wait()

            rv = rows_v.at[p]
            ov = ot_v.at[p]

            def hbody(h, _):
                h_vec = jnp.full((L,), h, jnp.int32)
                for bg in range(BBLK // L):
                    v = plsc.load_gather(rv, [iota + (bg * L), h_vec])
                    ov[h, pl.ds(bg * L, L)] = v
                return 0

            lax.fori_loop(0, HIDDEN, hbody, 0)

            pltpu.async_copy(
                ov,
                out_hbm.at[s, pl.ds(0, HIDDEN), pl.ds(b0, BBLK)],
                osem,
            )
            return 0

        lax.fori_loop(0, S, body, 0)

        for p in range(2):
            pltpu.make_async_copy(
                ot_v.at[p],
                out_hbm.at[0, pl.ds(0, HIDDEN), pl.ds(b0, BBLK)],
                osem,
            ).wait()

    out = emb(idx_t_host, table128)
    return jnp.transpose(out, (2, 0, 1))


# speedup vs baseline: 1.7370x; 1.7370x over previous
import functools

import jax
import jax.numpy as jnp
from jax import lax
from jax.experimental import pallas as pl
from jax.experimental.pallas import tpu as pltpu
from jax.experimental.pallas import tpu_sc as plsc

HIDDEN = 64
PADDED = 128
NC = 2
NS = 16
NW = NC * NS
CHUNK = 128


def kernel(input_ids, token_embedding_weight):
    B, S = input_ids.shape
    total = B * S
    per_w = total // NW
    n_chunks = per_w // CHUNK
    idx = input_ids.reshape(NW, n_chunks, CHUNK).astype(jnp.int32)
    table128 = jnp.pad(token_embedding_weight, ((0, 0), (0, PADDED - HIDDEN)))

    mesh = plsc.VectorSubcoreMesh(core_axis_name="c", subcore_axis_name="s")

    GPC = 2
    group = GPC * CHUNK
    n_groups = per_w // group

    @functools.partial(
        pl.kernel,
        mesh=mesh,
        out_type=jax.ShapeDtypeStruct((total, PADDED), jnp.float32),
        scratch_types=[
            pltpu.VMEM((n_chunks, CHUNK), jnp.int32),
            pltpu.VMEM((2, group, PADDED), jnp.float32),
            pltpu.SemaphoreType.DMA,
            pltpu.SemaphoreType.DMA,
            pltpu.SemaphoreType.DMA,
        ],
    )
    def emb(idx_hbm, table_hbm, out_hbm, idx_v, rows_v, gsem, osem0, osem1):
        wid = lax.axis_index("s") * NC + lax.axis_index("c")
        base = wid * per_w
        osems = (osem0, osem1)
        pltpu.sync_copy(idx_hbm.at[wid], idx_v)

        def fire(g, p):
            for b in range(GPC):
                pltpu.async_copy(
                    table_hbm.at[idx_v.at[g * GPC + b]],
                    rows_v.at[p, pl.ds(b * CHUNK, CHUNK)],
                    gsem,
                )

        fire(0, 0)

        def make_body(p):
            def body(g, _):
                for b in range(GPC):
                    pltpu.make_async_copy(
                        table_hbm.at[idx_v.at[g * GPC + b]],
                        rows_v.at[p, pl.ds(b * CHUNK, CHUNK)],
                        gsem,
                    ).wait()

                @pl.when(g >= 1)
                def _():
                    pltpu.make_async_copy(
                        rows_v.at[1 - p],
                        out_hbm.at[pl.ds(base, group)],
                        osems[1 - p],
                    ).wait()

                @pl.when(g + 1 < n_groups)
                def _():
                    fire(g + 1, 1 - p)

                pltpu.async_copy(
                    rows_v.at[p],
                    out_hbm.at[pl.ds(base + g * group, group)],
                    osems[p],
                )
                return 0

            return body

        body0 = make_body(0)
        body1 = make_body(1)

        def pair(h, _):
            body0(h * 2, 0)
            body1(h * 2 + 1, 0)
            return 0

        lax.fori_loop(0, n_groups // 2, pair, 0)

        last_sem = osems[(n_groups - 1) % 2]
        pltpu.make_async_copy(
            rows_v.at[(n_groups - 1) % 2],
            out_hbm.at[pl.ds(base, group)],
            last_sem,
        ).wait()

    out = emb(idx, table128)
    return out[:, :HIDDEN].reshape(B, S, HIDDEN)
